# Initial kernel scaffold; baseline (speedup 1.0000x reference)
import jax
import jax.numpy as jnp
from jax.experimental import pallas as pl


def kernel(x, emb0, emb1):
    raise NotImplementedError("write your pallas kernel here")


# speedup vs baseline: 2.2395x; 2.2395x over previous
import jax
import jax.numpy as jnp
from jax import lax
from jax.experimental import pallas as pl
from jax.experimental.pallas import tpu as pltpu, tpu_sc as plsc
import functools

_B, _T, _C = 2, 4096, 2048
_NC, _NS = 2, 16
_NW = _NC * _NS
_ROWS = _B * _T
_RPW = _ROWS // _NW
_CH = 16
_NCHUNK = _RPW // _CH
_L = 16
_G = _C // _L


def _body(x_hbm, e0_hbm, e1_hbm, out_hbm, x_buf, o_buf, e0_v, e1_v):
    wid = lax.axis_index("s") * _NC + lax.axis_index("c")
    start = wid * _RPW
    is_bstart = (start % _T) == 0

    pltpu.sync_copy(e0_hbm, e0_v)
    pltpu.sync_copy(e1_hbm, e1_v)

    def chunk_body(j, carry):
        cs = start + j * _CH
        first = jnp.logical_and(is_bstart, j == 0)

        @pl.when(first)
        def _():
            pltpu.sync_copy(x_hbm.at[pl.ds(cs * _C, _CH * _C)],
                            x_buf.at[pl.ds(_C, _CH * _C)])

        @pl.when(jnp.logical_not(first))
        def _():
            pltpu.sync_copy(x_hbm.at[pl.ds((cs - 1) * _C, (_CH + 1) * _C)],
                            x_buf)

        def g_body(g, carry):
            gb = g * _L
            e0s = e0_v[pl.ds(gb, _L)]
            e1s = e1_v[pl.ds(gb, _L)]
            for i in range(_CH):
                xs = x_buf[pl.ds(i * _C + gb, _L)]
                o_buf[pl.ds(i * _C + gb, _L)] = jnp.where(xs > 0, e1s, e0s)
            return carry

        lax.fori_loop(0, _G, g_body, 0, unroll=2)

        @pl.when(first)
        def _():
            def fix(g, carry):
                o_buf[pl.ds(g * _L, _L)] = e0_v[pl.ds(g * _L, _L)]
                return carry
            lax.fori_loop(0, _G, fix, 0)

        pltpu.sync_copy(o_buf, out_hbm.at[pl.ds(cs * _C, _CH * _C)])
        return carry

    lax.fori_loop(0, _NCHUNK, chunk_body, 0)


@functools.partial(jax.jit, static_argnames=())
def kernel(x, emb0, emb1):
    B, T, C = x.shape
    x_flat = x.reshape(B * T * C)
    e0 = emb0.reshape(C)
    e1 = emb1.reshape(C)

    mesh = plsc.VectorSubcoreMesh(core_axis_name="c", subcore_axis_name="s",
                                  num_cores=_NC, num_subcores=_NS)
    out_flat = pl.kernel(
        _body,
        out_type=jax.ShapeDtypeStruct((B * T * C,), jnp.float32),
        mesh=mesh,
        scratch_types=[
            pltpu.VMEM(((_CH + 1) * _C,), jnp.float32),
            pltpu.VMEM((_CH * _C,), jnp.float32),
            pltpu.VMEM((_C,), jnp.float32),
            pltpu.VMEM((_C,), jnp.float32),
        ],
    )(x_flat, e0, e1)
    return out_flat.reshape(B, T, C)
